# Initial kernel scaffold; baseline (speedup 1.0000x reference)
import jax
import jax.numpy as jnp
from jax.experimental import pallas as pl


def kernel(x, edge_index, W, b):
    raise NotImplementedError("write your pallas kernel here")


# speedup vs baseline: 3.7188x; 3.7188x over previous
import functools

import jax
import jax.numpy as jnp
from jax import lax
from jax.experimental import pallas as pl
from jax.experimental.pallas import tpu as pltpu
from jax.experimental.pallas import tpu_sc as plsc

N_NODES = 10000
D = 128
NC = 2
NS = 16
NW = NC * NS
CHUNK = 128
N_PAD = 10240
ZERO_ROWS = N_PAD // NS


def _sc_scatter(x, src, dst, zrows, zdeg):
    e_pad = src.shape[0]
    chunks_per_w = e_pad // (NW * CHUNK)
    mesh = plsc.VectorSubcoreMesh(core_axis_name="c", subcore_axis_name="s")

    @functools.partial(
        pl.kernel,
        out_type=[
            jax.ShapeDtypeStruct((NC * N_PAD, D), jnp.float32),
            jax.ShapeDtypeStruct((NC * N_PAD,), jnp.float32),
        ],
        mesh=mesh,
        scratch_types=[
            pltpu.VMEM((CHUNK,), jnp.int32),
            pltpu.VMEM((CHUNK,), jnp.int32),
            pltpu.VMEM((CHUNK, D), jnp.float32),
            pltpu.VMEM((CHUNK,), jnp.float32),
            pltpu.VMEM_SHARED((N_PAD, D), jnp.float32),
            pltpu.VMEM_SHARED((N_PAD,), jnp.float32),
            pltpu.SemaphoreType.DMA,
        ],
    )
    def k(x_hbm, src_hbm, dst_hbm, zr_hbm, zd_hbm, agg_out, deg_out,
          sidx, didx, rows, ones, agg_sh, deg_sh, sem):
        c = lax.axis_index("c")
        s = lax.axis_index("s")
        wid = s * NC + c

        pltpu.sync_copy(zr_hbm.at[pl.ds(s * ZERO_ROWS, ZERO_ROWS)],
                        agg_sh.at[pl.ds(s * ZERO_ROWS, ZERO_ROWS)])

        @pl.when(s == 0)
        def _():
            pltpu.sync_copy(zd_hbm, deg_sh)

        for j in range(CHUNK // 16):
            ones[pl.ds(j * 16, 16)] = jnp.ones((16,), jnp.float32)

        plsc.subcore_barrier()

        base = wid * (chunks_per_w * CHUNK)

        def body(g, carry):
            off = base + g * CHUNK
            pltpu.sync_copy(src_hbm.at[pl.ds(off, CHUNK)], sidx)
            pltpu.sync_copy(dst_hbm.at[pl.ds(off, CHUNK)], didx)
            pltpu.async_copy(x_hbm.at[sidx], rows, sem).wait()
            pltpu.sync_copy(rows, agg_sh.at[didx], add=True)
            pltpu.sync_copy(ones, deg_sh.at[didx], add=True)
            return carry

        lax.fori_loop(0, chunks_per_w, body, 0)

        plsc.subcore_barrier()

        pltpu.sync_copy(agg_sh.at[pl.ds(s * ZERO_ROWS, ZERO_ROWS)],
                        agg_out.at[pl.ds(c * N_PAD + s * ZERO_ROWS, ZERO_ROWS)])

        @pl.when(s == 0)
        def _():
            pltpu.sync_copy(deg_sh, deg_out.at[pl.ds(c * N_PAD, N_PAD)])

    return k(x, src, dst, zrows, zdeg)


def _tc_finish(agg, deg, W, b):
    blk = 1024
    grid = (N_PAD // blk,)

    def body(a_ref, d_ref, w_ref, b_ref, o_ref):
        a = a_ref[0] + a_ref[1]
        dg = d_ref[0] + d_ref[1]
        o_ref[...] = (jnp.dot(a, w_ref[...], preferred_element_type=jnp.float32)
                      + dg[:, None] * b_ref[...])

    return pl.pallas_call(
        body,
        grid=grid,
        in_specs=[
            pl.BlockSpec((NC, blk, D), lambda i: (0, i, 0)),
            pl.BlockSpec((NC, blk), lambda i: (0, i)),
            pl.BlockSpec((D, D), lambda i: (0, 0)),
            pl.BlockSpec((1, D), lambda i: (0, 0)),
        ],
        out_specs=pl.BlockSpec((blk, D), lambda i: (i, 0)),
        out_shape=jax.ShapeDtypeStruct((N_PAD, D), jnp.float32),
    )(agg, deg, W, b.reshape(1, D))


def kernel(x, edge_index, W, b):
    e = edge_index.astype(jnp.int32)
    src, dst = e[0], e[1]
    n_edges = src.shape[0]
    e_pad = ((n_edges + NW * CHUNK - 1) // (NW * CHUNK)) * (NW * CHUNK)
    pad = e_pad - n_edges
    src = jnp.concatenate([src, jnp.zeros((pad,), jnp.int32)])
    dst = jnp.concatenate([dst, jnp.full((pad,), N_NODES, jnp.int32)])
    zrows = jnp.zeros((N_PAD, D), jnp.float32)
    zdeg = jnp.zeros((N_PAD,), jnp.float32)
    agg, deg = _sc_scatter(x, src, dst, zrows, zdeg)
    out = _tc_finish(agg.reshape(NC, N_PAD, D), deg.reshape(NC, N_PAD), W, b)
    return out[:N_NODES]
